# Initial kernel scaffold; baseline (speedup 1.0000x reference)
import jax
import jax.numpy as jnp
from jax.experimental import pallas as pl


def kernel(input, embed_weight, new_embed_weight):
    raise NotImplementedError("write your pallas kernel here")


# speedup vs baseline: 6.0173x; 6.0173x over previous
import functools

import jax
import jax.numpy as jnp
from jax import lax
from jax.experimental import pallas as pl
from jax.experimental.pallas import tpu as pltpu
from jax.experimental.pallas import tpu_sc as plsc

B_ROWS = 4096
T_COLS = 200
D = 128
B_TOTAL = B_ROWS * T_COLS
NC = 2
NS = 16
NW = NC * NS
ROWS_PER_W = B_TOTAL // NW
CH = 128
CHUNKS_PER_W = ROWS_PER_W // CH


def _gather_body(table_hbm, idx_hbm, out_hbm, idx_v, rows_v, sem):
    c = lax.axis_index("c")
    s = lax.axis_index("s")
    wid = s * NC + c
    base_chunk = wid * CHUNKS_PER_W
    pltpu.sync_copy(idx_hbm.at[pl.ds(base_chunk, CHUNKS_PER_W)], idx_v)

    def body(g, carry):
        pltpu.async_copy(table_hbm.at[idx_v.at[g]], rows_v, sem).wait()
        pltpu.sync_copy(rows_v, out_hbm.at[pl.ds((base_chunk + g) * CH, CH)])
        return carry

    lax.fori_loop(0, CHUNKS_PER_W, body, 0)


@jax.jit
def kernel(input, embed_weight, new_embed_weight):
    table = jnp.concatenate([embed_weight, new_embed_weight], axis=0)
    idx = input.reshape(-1).astype(jnp.int32).reshape(B_TOTAL // CH, CH)
    mesh = plsc.VectorSubcoreMesh(core_axis_name="c", subcore_axis_name="s")
    run = pl.kernel(
        _gather_body,
        out_type=jax.ShapeDtypeStruct((B_TOTAL, D), jnp.float32),
        mesh=mesh,
        scratch_types=[
            pltpu.VMEM((CHUNKS_PER_W, CH), jnp.int32),
            pltpu.VMEM((CH, D), jnp.float32),
            pltpu.SemaphoreType.DMA,
        ],
    )
    out = run(table, idx)
    return out.reshape(B_ROWS, T_COLS, D)


# speedup vs baseline: 8.5047x; 1.4134x over previous
import functools

import jax
import jax.numpy as jnp
from jax import lax
from jax.experimental import pallas as pl
from jax.experimental.pallas import tpu as pltpu
from jax.experimental.pallas import tpu_sc as plsc

B_ROWS = 4096
T_COLS = 200
D = 128
B_TOTAL = B_ROWS * T_COLS
NC = 2
NS = 16
NW = NC * NS
ROWS_PER_W = B_TOTAL // NW
CH = 128
CHUNKS_PER_W = ROWS_PER_W // CH


def _gather_body(table_hbm, idx_hbm, out_hbm, idx_v, buf0, buf1,
                 gsem0, gsem1, ssem0, ssem1):
    c = lax.axis_index("c")
    s = lax.axis_index("s")
    wid = s * NC + c
    base_chunk = wid * CHUNKS_PER_W
    pltpu.sync_copy(idx_hbm.at[pl.ds(base_chunk, CHUNKS_PER_W)], idx_v)

    def gstart(g, buf, sem):
        pltpu.async_copy(table_hbm.at[idx_v.at[g]], buf, sem)

    def gwait(buf, sem):
        pltpu.make_async_copy(table_hbm.at[idx_v.at[0]], buf, sem).wait()

    def sstart(g, buf, sem):
        pltpu.async_copy(buf, out_hbm.at[pl.ds((base_chunk + g) * CH, CH)], sem)

    def swait(buf, sem):
        pltpu.make_async_copy(buf, out_hbm.at[pl.ds(0, CH)], sem).wait()

    gstart(0, buf0, gsem0)
    gstart(1, buf1, gsem1)
    gwait(buf0, gsem0)
    sstart(0, buf0, ssem0)
    swait(buf0, ssem0)
    gstart(2, buf0, gsem0)
    gwait(buf1, gsem1)
    sstart(1, buf1, ssem1)

    def body(h, carry):
        g0 = 2 * h
        swait(buf1, ssem1)
        gstart(g0 + 1, buf1, gsem1)
        gwait(buf0, gsem0)
        sstart(g0, buf0, ssem0)
        swait(buf0, ssem0)
        gstart(g0 + 2, buf0, gsem0)
        gwait(buf1, gsem1)
        sstart(g0 + 1, buf1, ssem1)
        return carry

    lax.fori_loop(1, CHUNKS_PER_W // 2 - 1, body, 0)

    last = CHUNKS_PER_W - 2
    swait(buf1, ssem1)
    gstart(last + 1, buf1, gsem1)
    gwait(buf0, gsem0)
    sstart(last, buf0, ssem0)
    swait(buf0, ssem0)
    gwait(buf1, gsem1)
    sstart(last + 1, buf1, ssem1)
    swait(buf1, ssem1)


@jax.jit
def kernel(input, embed_weight, new_embed_weight):
    table = jnp.concatenate([embed_weight, new_embed_weight], axis=0)
    idx = input.reshape(-1).astype(jnp.int32).reshape(B_TOTAL // CH, CH)
    mesh = plsc.VectorSubcoreMesh(core_axis_name="c", subcore_axis_name="s")
    run = pl.kernel(
        _gather_body,
        out_type=jax.ShapeDtypeStruct((B_TOTAL, D), jnp.float32),
        mesh=mesh,
        scratch_types=[
            pltpu.VMEM((CHUNKS_PER_W, CH), jnp.int32),
            pltpu.VMEM((CH, D), jnp.float32),
            pltpu.VMEM((CH, D), jnp.float32),
            pltpu.SemaphoreType.DMA,
            pltpu.SemaphoreType.DMA,
            pltpu.SemaphoreType.DMA,
            pltpu.SemaphoreType.DMA,
        ],
    )
    out = run(table, idx)
    return out.reshape(B_ROWS, T_COLS, D)
